# baseline (speedup 1.0000x reference)
import functools

import jax
import jax.numpy as jnp
from jax import lax
from jax.experimental import pallas as pl
from jax.experimental.pallas import tpu as pltpu
from jax.experimental.pallas import tpu_sc as plsc

N = 10000
E = 320000
D_IN = 128
D_HID = 16
D_OUT = 7

NC = 2
NS = 16
NW = NC * NS
N_PAD = 10240
ROWS_PER_SUB = N_PAD // NS
B = 128
NBUF = 10
K = 80
SPLIT = (NW - 1) * K * B
TAIL_R = (E - SPLIT) // B
RM = 2000
RO = 2000


@functools.cache
def _sc_kernels():
    mesh = plsc.VectorSubcoreMesh(core_axis_name="c", subcore_axis_name="s")

    def _stage_edges(w, a31_hbm, at_hbm, apad_hbm, a_v):
        @pl.when(w < NW - 1)
        def _():
            pltpu.sync_copy(a31_hbm.at[w], a_v)

        @pl.when(w == NW - 1)
        def _():
            pltpu.sync_copy(at_hbm, a_v.at[pl.ds(0, TAIL_R)])
            pltpu.sync_copy(apad_hbm, a_v.at[pl.ds(TAIL_R, K - TAIL_R)])

    @functools.partial(
        pl.kernel,
        mesh=mesh,
        out_type=jax.ShapeDtypeStruct((NC, N_PAD), jnp.float32),
        compiler_params=pltpu.CompilerParams(use_tc_tiling_on_sc=False),
        scratch_types=[
            pltpu.VMEM((K, B), jnp.int32),
            pltpu.VMEM((B,), jnp.float32),
            pltpu.VMEM_SHARED((N_PAD,), jnp.float32),
            pltpu.SemaphoreType.DMA,
        ],
    )
    def _sc_degree(dst31_hbm, dstt_hbm, padn_hbm, zeros1_hbm, out_hbm,
                   dst_v, ones_v, acc, sem):
        c = lax.axis_index("c")
        s = lax.axis_index("s")
        w = c * NS + s
        _stage_edges(w, dst31_hbm, dstt_hbm, padn_hbm, dst_v)
        for i in range(B // 16):
            ones_v[pl.ds(i * 16, 16)] = jnp.full((16,), 1.0, jnp.float32)
        pltpu.sync_copy(zeros1_hbm.at[pl.ds(s * ROWS_PER_SUB, ROWS_PER_SUB)],
                        acc.at[pl.ds(s * ROWS_PER_SUB, ROWS_PER_SUB)])
        plsc.subcore_barrier()

        def fire(k, carry):
            pltpu.async_copy(ones_v, acc.at[dst_v.at[k]], sem, add=True)
            return carry

        lax.fori_loop(0, K, fire, 0)

        def drain(k, carry):
            pltpu.make_async_copy(ones_v, acc.at[dst_v.at[0]], sem).wait()
            return carry

        lax.fori_loop(0, K, drain, 0)
        plsc.subcore_barrier()
        pltpu.sync_copy(acc.at[pl.ds(s * ROWS_PER_SUB, ROWS_PER_SUB)],
                        out_hbm.at[c, pl.ds(s * ROWS_PER_SUB, ROWS_PER_SUB)])

    def _ring(src_v, dst_v, rows_v, acc, hsh, gsem, ssem):

        G = K // NBUF
        for b in range(NBUF):
            pltpu.async_copy(hsh.at[src_v.at[b]], rows_v.at[0, b],
                             gsem.at[b])

        def pair(p, carry):
            for half in (0, 1):
                g = 2 * p + half
                for b in range(NBUF):
                    k = g * NBUF + b
                    pltpu.make_async_copy(hsh.at[src_v.at[k]],
                                          rows_v.at[half, b],
                                          gsem.at[b]).wait()

                    @pl.when(g > 0)
                    def _():
                        pltpu.make_async_copy(rows_v.at[1 - half, b],
                                              acc.at[dst_v.at[k]],
                                              ssem.at[b]).wait()

                    pltpu.async_copy(rows_v.at[half, b],
                                     acc.at[dst_v.at[k]],
                                     ssem.at[b], add=True)

                    @pl.when(g < G - 1)
                    def _():
                        pltpu.async_copy(hsh.at[src_v.at[k + NBUF]],
                                         rows_v.at[1 - half, b],
                                         gsem.at[b])
            return carry

        lax.fori_loop(0, G // 2, pair, 0)
        for b in range(NBUF):
            pltpu.make_async_copy(rows_v.at[1, b], acc.at[dst_v.at[b]],
                                  ssem.at[b]).wait()

    CH = N // NS
    _agg_scratch = [
        pltpu.VMEM((K, B), jnp.int32),
        pltpu.VMEM((K, B), jnp.int32),
        pltpu.VMEM((2, NBUF, B, D_HID), jnp.float32),
        pltpu.VMEM_SHARED((N_PAD, D_HID), jnp.float32),
        pltpu.VMEM_SHARED((N, D_HID), jnp.float32),
        pltpu.SemaphoreType.DMA((NBUF,)),
        pltpu.SemaphoreType.DMA((NBUF,)),
    ]

    @functools.partial(
        pl.kernel,
        mesh=mesh,
        out_type=jax.ShapeDtypeStruct((NC, N_PAD, D_HID), jnp.float32),
        compiler_params=pltpu.CompilerParams(use_tc_tiling_on_sc=False),
        scratch_types=_agg_scratch,
    )
    def _sc_aggregate1(h_hbm, src31_hbm, srct_hbm, pad0_hbm,
                       dst31_hbm, dstt_hbm, padn_hbm, zeros2_hbm, out_hbm,
                       src_v, dst_v, rows_v, acc, hsh, gsem, ssem):
        c = lax.axis_index("c")
        s = lax.axis_index("s")
        w = c * NS + s
        _stage_edges(w, src31_hbm, srct_hbm, pad0_hbm, src_v)
        _stage_edges(w, dst31_hbm, dstt_hbm, padn_hbm, dst_v)
        pltpu.sync_copy(h_hbm.at[pl.ds(s * CH, CH)], hsh.at[pl.ds(s * CH, CH)])
        @pl.when(c == 0)
        def _():
            pltpu.sync_copy(h_hbm.at[pl.ds(s * CH, CH)],
                            acc.at[pl.ds(s * CH, CH)])

        @pl.when(c == 1)
        def _():
            pltpu.sync_copy(zeros2_hbm.at[pl.ds(s * CH, CH)],
                            acc.at[pl.ds(s * CH, CH)])

        @pl.when(s == NS - 1)
        def _():
            pltpu.sync_copy(zeros2_hbm.at[pl.ds(N, N_PAD - N)],
                            acc.at[pl.ds(N, N_PAD - N)])

        plsc.subcore_barrier()
        _ring(src_v, dst_v, rows_v, acc, hsh, gsem, ssem)
        plsc.subcore_barrier()
        pltpu.sync_copy(acc.at[pl.ds(s * ROWS_PER_SUB, ROWS_PER_SUB)],
                        out_hbm.at[c, pl.ds(s * ROWS_PER_SUB, ROWS_PER_SUB)])

    @functools.partial(
        pl.kernel,
        mesh=mesh,
        out_type=jax.ShapeDtypeStruct((NC, N_PAD, D_HID), jnp.float32),
        compiler_params=pltpu.CompilerParams(use_tc_tiling_on_sc=False),
        scratch_types=_agg_scratch + [
            pltpu.VMEM((CH, D_HID), jnp.float32),
            pltpu.VMEM((CH, D_HID), jnp.float32),
            pltpu.VMEM((CH, D_HID), jnp.float32),
            pltpu.VMEM((CH, D_HID), jnp.float32),
            pltpu.VMEM((D_HID,), jnp.float32),
        ],
    )
    def _sc_aggregate2(s1_hbm, dinv_hbm, b1_hbm, src31_hbm, srct_hbm,
                       pad0_hbm, dst31_hbm, dstt_hbm, padn_hbm,
                       zeros2_hbm, out_hbm,
                       src_v, dst_v, rows_v, acc, hsh, gsem, ssem,
                       a_v, bvec_v, d_v, h2p_v, b1_v):
        c = lax.axis_index("c")
        s = lax.axis_index("s")
        w = c * NS + s
        _stage_edges(w, src31_hbm, srct_hbm, pad0_hbm, src_v)
        _stage_edges(w, dst31_hbm, dstt_hbm, padn_hbm, dst_v)
        base = s * CH
        pltpu.sync_copy(s1_hbm.at[0, pl.ds(base, CH)], a_v)
        pltpu.sync_copy(s1_hbm.at[1, pl.ds(base, CH)], bvec_v)
        pltpu.sync_copy(dinv_hbm.at[pl.ds(base, CH)], d_v)
        pltpu.sync_copy(b1_hbm, b1_v)

        def mkrow(i, carry):
            d = d_v[i]
            agg = d * (a_v[i] + bvec_v[i]) + b1_v[...]
            h2p_v[i] = d * jnp.maximum(agg, 0.0)
            return carry

        lax.fori_loop(0, CH, mkrow, 0)
        pltpu.sync_copy(h2p_v, hsh.at[pl.ds(base, CH)])

        @pl.when(c == 0)
        def _():
            pltpu.sync_copy(h2p_v, acc.at[pl.ds(base, CH)])

        @pl.when(c == 1)
        def _():
            pltpu.sync_copy(zeros2_hbm.at[pl.ds(base, CH)],
                            acc.at[pl.ds(base, CH)])

        @pl.when(s == NS - 1)
        def _():
            pltpu.sync_copy(zeros2_hbm.at[pl.ds(N, N_PAD - N)],
                            acc.at[pl.ds(N, N_PAD - N)])

        plsc.subcore_barrier()
        _ring(src_v, dst_v, rows_v, acc, hsh, gsem, ssem)
        plsc.subcore_barrier()
        pltpu.sync_copy(acc.at[pl.ds(s * ROWS_PER_SUB, ROWS_PER_SUB)],
                        out_hbm.at[c, pl.ds(s * ROWS_PER_SUB, ROWS_PER_SUB)])

    return _sc_degree, _sc_aggregate1, _sc_aggregate2


def _tc_mm_body(x_ref, w1_ref, h_ref):
    h_ref[...] = jnp.dot(x_ref[...], w1_ref[...],
                         preferred_element_type=jnp.float32)


def _tc_mm(x, w1):
    return pl.pallas_call(
        _tc_mm_body,
        grid=(N // RM,),
        in_specs=[
            pl.BlockSpec((RM, D_IN), lambda i: (i, 0)),
            pl.BlockSpec((D_IN, D_HID), lambda i: (0, 0)),
        ],
        out_specs=pl.BlockSpec((RM, D_HID), lambda i: (i, 0)),
        out_shape=jax.ShapeDtypeStruct((N, D_HID), jnp.float32),
    )(x, w1)


def _tc_scale_body(deg_ref, h_ref, h1p_ref, dinv_ref):
    deg2 = deg_ref[...]
    deg = deg2[:, 0:1] + deg2[:, 1:2] + 1.0
    dinv = lax.rsqrt(deg)
    h1p_ref[...] = h_ref[...] * dinv
    dinv_ref[...] = jnp.broadcast_to(dinv, (N, D_HID))


def _tc_scale(deg_t, h):
    return pl.pallas_call(
        _tc_scale_body,
        grid=(1,),
        in_specs=[
            pl.BlockSpec((N, NC), lambda i: (0, 0)),
            pl.BlockSpec((N, D_HID), lambda i: (0, 0)),
        ],
        out_specs=[
            pl.BlockSpec((N, D_HID), lambda i: (0, 0)),
            pl.BlockSpec((N, D_HID), lambda i: (0, 0)),
        ],
        out_shape=[
            jax.ShapeDtypeStruct((N, D_HID), jnp.float32),
            jax.ShapeDtypeStruct((N, D_HID), jnp.float32),
        ],
    )(deg_t, h)


def _tc_out_body(s_ref, dinv_ref, w2_ref, b2_ref, out_ref):
    sb = s_ref[...]
    agg = dinv_ref[...] * (sb[0] + sb[1])
    out_ref[...] = jnp.dot(agg, w2_ref[...],
                           preferred_element_type=jnp.float32) + b2_ref[...]


def _tc_out(s2, dinvb, w2, b2):
    return pl.pallas_call(
        _tc_out_body,
        grid=(N // RO,),
        in_specs=[
            pl.BlockSpec((NC, RO, D_HID), lambda i: (0, i, 0)),
            pl.BlockSpec((RO, D_HID), lambda i: (i, 0)),
            pl.BlockSpec((D_HID, D_OUT), lambda i: (0, 0)),
            pl.BlockSpec((1, D_OUT), lambda i: (0, 0)),
        ],
        out_specs=pl.BlockSpec((RO, D_OUT), lambda i: (i, 0)),
        out_shape=jax.ShapeDtypeStruct((N, D_OUT), jnp.float32),
    )(s2, dinvb, w2, b2.reshape(1, D_OUT))


def kernel(x, edge_index, W1, b1, W2, b2):
    src = edge_index[0]
    dst = edge_index[1]
    src31 = src[:SPLIT].reshape(NW - 1, K, B)
    dst31 = dst[:SPLIT].reshape(NW - 1, K, B)
    srct = src[SPLIT:].reshape(TAIL_R, B)
    dstt = dst[SPLIT:].reshape(TAIL_R, B)
    pad0 = jnp.zeros((K - TAIL_R, B), jnp.int32)
    padn = jnp.full((K - TAIL_R, B), N, jnp.int32)
    zeros1 = jnp.zeros((N_PAD,), jnp.float32)
    zeros2 = jnp.zeros((N_PAD, D_HID), jnp.float32)

    _sc_degree, _sc_aggregate1, _sc_aggregate2 = _sc_kernels()
    degp = _sc_degree(dst31, dstt, padn, zeros1)
    h = _tc_mm(x, W1)
    h1p, dinvb = _tc_scale(degp.T, h)
    s1 = _sc_aggregate1(h1p, src31, srct, pad0, dst31, dstt, padn, zeros2)
    s2 = _sc_aggregate2(s1, dinvb, b1, src31, srct, pad0, dst31, dstt, padn,
                        zeros2)
    out = _tc_out(s2, dinvb, W2, b2)
    return out


# speedup vs baseline: 1.0492x; 1.0492x over previous
import functools

import jax
import jax.numpy as jnp
from jax import lax
from jax.experimental import pallas as pl
from jax.experimental.pallas import tpu as pltpu
from jax.experimental.pallas import tpu_sc as plsc

N = 10000
E = 320000
D_IN = 128
D_HID = 16
D_OUT = 7

NC = 2
NS = 16
NW = NC * NS
N_PAD = 10240
ROWS_PER_SUB = N_PAD // NS
B = 128
E_PER_W = E // NW
NBUF = 10
K = 80
R = 1000


@functools.cache
def _sc_kernels():
    mesh = plsc.VectorSubcoreMesh(core_axis_name="c", subcore_axis_name="s")

    @functools.partial(
        pl.kernel,
        mesh=mesh,
        out_type=jax.ShapeDtypeStruct((NC, N_PAD), jnp.float32),
        compiler_params=pltpu.CompilerParams(use_tc_tiling_on_sc=False),
        scratch_types=[
            pltpu.VMEM((K, B), jnp.int32),
            pltpu.VMEM((B,), jnp.float32),
            pltpu.VMEM_SHARED((N_PAD,), jnp.float32),
            pltpu.SemaphoreType.DMA,
        ],
    )
    def _sc_degree(dst_hbm, zeros1_hbm, out_hbm, dst_v, ones_v, acc, sem):
        c = lax.axis_index("c")
        s = lax.axis_index("s")
        w = c * NS + s
        pltpu.sync_copy(dst_hbm.at[w], dst_v)
        for i in range(B // 16):
            ones_v[pl.ds(i * 16, 16)] = jnp.full((16,), 1.0, jnp.float32)
        pltpu.sync_copy(zeros1_hbm.at[pl.ds(s * ROWS_PER_SUB, ROWS_PER_SUB)],
                        acc.at[pl.ds(s * ROWS_PER_SUB, ROWS_PER_SUB)])
        plsc.subcore_barrier()

        def fire(k, carry):
            pltpu.async_copy(ones_v, acc.at[dst_v.at[k]], sem, add=True)
            return carry

        lax.fori_loop(0, K, fire, 0)

        def drain(k, carry):
            pltpu.make_async_copy(ones_v, acc.at[dst_v.at[0]], sem).wait()
            return carry

        lax.fori_loop(0, K, drain, 0)
        plsc.subcore_barrier()
        pltpu.sync_copy(acc.at[pl.ds(s * ROWS_PER_SUB, ROWS_PER_SUB)],
                        out_hbm.at[c, pl.ds(s * ROWS_PER_SUB, ROWS_PER_SUB)])

    def _ring(src_v, dst_v, rows_v, acc, hsh, gsem, ssem):

        G = K // NBUF
        for b in range(NBUF):
            pltpu.async_copy(hsh.at[src_v.at[b]], rows_v.at[0, b],
                             gsem.at[b])

        def pair(p, carry):
            for half in (0, 1):
                g = 2 * p + half
                for b in range(NBUF):
                    k = g * NBUF + b
                    pltpu.make_async_copy(hsh.at[src_v.at[k]],
                                          rows_v.at[half, b],
                                          gsem.at[b]).wait()

                    @pl.when(g > 0)
                    def _():
                        pltpu.make_async_copy(rows_v.at[1 - half, b],
                                              acc.at[dst_v.at[k]],
                                              ssem.at[b]).wait()

                    pltpu.async_copy(rows_v.at[half, b],
                                     acc.at[dst_v.at[k]],
                                     ssem.at[b], add=True)

                    @pl.when(g < G - 1)
                    def _():
                        pltpu.async_copy(hsh.at[src_v.at[k + NBUF]],
                                         rows_v.at[1 - half, b],
                                         gsem.at[b])
            return carry

        lax.fori_loop(0, G // 2, pair, 0)
        for b in range(NBUF):
            pltpu.make_async_copy(rows_v.at[1, b], acc.at[dst_v.at[b]],
                                  ssem.at[b]).wait()

    CH = N // NS
    _agg_scratch = [
        pltpu.VMEM((K, B), jnp.int32),
        pltpu.VMEM((K, B), jnp.int32),
        pltpu.VMEM((2, NBUF, B, D_HID), jnp.float32),
        pltpu.VMEM_SHARED((N_PAD, D_HID), jnp.float32),
        pltpu.VMEM_SHARED((N, D_HID), jnp.float32),
        pltpu.SemaphoreType.DMA((NBUF,)),
        pltpu.SemaphoreType.DMA((NBUF,)),
    ]

    @functools.partial(
        pl.kernel,
        mesh=mesh,
        out_type=jax.ShapeDtypeStruct((NC, N_PAD, D_HID), jnp.float32),
        compiler_params=pltpu.CompilerParams(use_tc_tiling_on_sc=False),
        scratch_types=_agg_scratch,
    )
    def _sc_aggregate1(h_hbm, src_hbm, dst_hbm, zeros2_hbm, out_hbm,
                       src_v, dst_v, rows_v, acc, hsh, gsem, ssem):
        c = lax.axis_index("c")
        s = lax.axis_index("s")
        w = c * NS + s
        pltpu.sync_copy(src_hbm.at[w], src_v)
        pltpu.sync_copy(dst_hbm.at[w], dst_v)
        pltpu.sync_copy(h_hbm.at[pl.ds(s * CH, CH)], hsh.at[pl.ds(s * CH, CH)])
        @pl.when(c == 0)
        def _():
            pltpu.sync_copy(h_hbm.at[pl.ds(s * CH, CH)],
                            acc.at[pl.ds(s * CH, CH)])

        @pl.when(c == 1)
        def _():
            pltpu.sync_copy(zeros2_hbm.at[pl.ds(s * CH, CH)],
                            acc.at[pl.ds(s * CH, CH)])

        @pl.when(s == NS - 1)
        def _():
            pltpu.sync_copy(zeros2_hbm.at[pl.ds(N, N_PAD - N)],
                            acc.at[pl.ds(N, N_PAD - N)])

        plsc.subcore_barrier()
        _ring(src_v, dst_v, rows_v, acc, hsh, gsem, ssem)
        plsc.subcore_barrier()
        pltpu.sync_copy(acc.at[pl.ds(s * ROWS_PER_SUB, ROWS_PER_SUB)],
                        out_hbm.at[c, pl.ds(s * ROWS_PER_SUB, ROWS_PER_SUB)])

    @functools.partial(
        pl.kernel,
        mesh=mesh,
        out_type=jax.ShapeDtypeStruct((NC, N_PAD, D_HID), jnp.float32),
        compiler_params=pltpu.CompilerParams(use_tc_tiling_on_sc=False),
        scratch_types=_agg_scratch + [
            pltpu.VMEM((CH, D_HID), jnp.float32),
            pltpu.VMEM((CH, D_HID), jnp.float32),
            pltpu.VMEM((CH, D_HID), jnp.float32),
            pltpu.VMEM((CH, D_HID), jnp.float32),
            pltpu.VMEM((D_HID,), jnp.float32),
        ],
    )
    def _sc_aggregate2(s1_hbm, dinv_hbm, b1_hbm, src_hbm, dst_hbm,
                       zeros2_hbm, out_hbm,
                       src_v, dst_v, rows_v, acc, hsh, gsem, ssem,
                       a_v, bvec_v, d_v, h2p_v, b1_v):
        c = lax.axis_index("c")
        s = lax.axis_index("s")
        w = c * NS + s
        pltpu.sync_copy(src_hbm.at[w], src_v)
        pltpu.sync_copy(dst_hbm.at[w], dst_v)
        base = s * CH
        pltpu.sync_copy(s1_hbm.at[0, pl.ds(base, CH)], a_v)
        pltpu.sync_copy(s1_hbm.at[1, pl.ds(base, CH)], bvec_v)
        pltpu.sync_copy(dinv_hbm.at[pl.ds(base, CH)], d_v)
        pltpu.sync_copy(b1_hbm, b1_v)

        def mkrow(i, carry):
            d = d_v[i]
            agg = d * (a_v[i] + bvec_v[i]) + b1_v[...]
            h2p_v[i] = d * jnp.maximum(agg, 0.0)
            return carry

        lax.fori_loop(0, CH, mkrow, 0)
        pltpu.sync_copy(h2p_v, hsh.at[pl.ds(base, CH)])

        @pl.when(c == 0)
        def _():
            pltpu.sync_copy(h2p_v, acc.at[pl.ds(base, CH)])

        @pl.when(c == 1)
        def _():
            pltpu.sync_copy(zeros2_hbm.at[pl.ds(base, CH)],
                            acc.at[pl.ds(base, CH)])

        @pl.when(s == NS - 1)
        def _():
            pltpu.sync_copy(zeros2_hbm.at[pl.ds(N, N_PAD - N)],
                            acc.at[pl.ds(N, N_PAD - N)])

        plsc.subcore_barrier()
        _ring(src_v, dst_v, rows_v, acc, hsh, gsem, ssem)
        plsc.subcore_barrier()
        pltpu.sync_copy(acc.at[pl.ds(s * ROWS_PER_SUB, ROWS_PER_SUB)],
                        out_hbm.at[c, pl.ds(s * ROWS_PER_SUB, ROWS_PER_SUB)])

    return _sc_degree, _sc_aggregate1, _sc_aggregate2


def _tc_first_body(deg_ref, x_ref, w1_ref, h1p_ref, dinv_ref):
    deg2 = deg_ref[...]
    deg = deg2[:, 0:1] + deg2[:, 1:2] + 1.0
    dinv = lax.rsqrt(deg)
    h = jnp.dot(x_ref[...], w1_ref[...],
                preferred_element_type=jnp.float32)
    h1p_ref[...] = h * dinv
    dinv_ref[...] = jnp.broadcast_to(dinv, (R, D_HID))


def _tc_first(deg_t, x, w1):
    return pl.pallas_call(
        _tc_first_body,
        grid=(N // R,),
        in_specs=[
            pl.BlockSpec((R, NC), lambda i: (i, 0)),
            pl.BlockSpec((R, D_IN), lambda i: (i, 0)),
            pl.BlockSpec((D_IN, D_HID), lambda i: (0, 0)),
        ],
        out_specs=[
            pl.BlockSpec((R, D_HID), lambda i: (i, 0)),
            pl.BlockSpec((R, D_HID), lambda i: (i, 0)),
        ],
        out_shape=[
            jax.ShapeDtypeStruct((N, D_HID), jnp.float32),
            jax.ShapeDtypeStruct((N, D_HID), jnp.float32),
        ],
    )(deg_t, x, w1)


def _tc_out_body(s_ref, dinv_ref, w2_ref, b2_ref, out_ref):
    sb = s_ref[...]
    agg = dinv_ref[...] * (sb[0] + sb[1])
    out_ref[...] = jnp.dot(agg, w2_ref[...],
                           preferred_element_type=jnp.float32) + b2_ref[...]


def _tc_out(s2, dinvb, w2, b2):
    return pl.pallas_call(
        _tc_out_body,
        grid=(N // R,),
        in_specs=[
            pl.BlockSpec((NC, R, D_HID), lambda i: (0, i, 0)),
            pl.BlockSpec((R, D_HID), lambda i: (i, 0)),
            pl.BlockSpec((D_HID, D_OUT), lambda i: (0, 0)),
            pl.BlockSpec((1, D_OUT), lambda i: (0, 0)),
        ],
        out_specs=pl.BlockSpec((R, D_OUT), lambda i: (i, 0)),
        out_shape=jax.ShapeDtypeStruct((N, D_OUT), jnp.float32),
    )(s2, dinvb, w2, b2.reshape(1, D_OUT))


def _pad_edges(idx, fill):
    per_w = idx.reshape(NW, E_PER_W)
    pad = jnp.full((NW, K * B - E_PER_W), fill, jnp.int32)
    return jnp.concatenate([per_w, pad], axis=1).reshape(NW, K, B)


def kernel(x, edge_index, W1, b1, W2, b2):
    src = edge_index[0]
    dst = edge_index[1]
    srcp = _pad_edges(src, 0)
    dstp = _pad_edges(dst, N)
    zeros1 = jnp.zeros((N_PAD,), jnp.float32)
    zeros2 = jnp.zeros((N_PAD, D_HID), jnp.float32)

    _sc_degree, _sc_aggregate1, _sc_aggregate2 = _sc_kernels()
    degp = _sc_degree(dstp, zeros1)
    h1p, dinvb = _tc_first(degp.T, x, W1)
    s1 = _sc_aggregate1(h1p, srcp, dstp, zeros2)
    s2 = _sc_aggregate2(s1, dinvb, b1, srcp, dstp, zeros2)
    out = _tc_out(s2, dinvb, W2, b2)
    return out
